# Initial kernel scaffold; baseline (speedup 1.0000x reference)
import jax
import jax.numpy as jnp
from jax.experimental import pallas as pl


def kernel(x, edge_index, edge_attr, W1, b1, W2, b2, diff_param):
    raise NotImplementedError("write your pallas kernel here")


# speedup vs baseline: 16.0486x; 16.0486x over previous
import functools

import jax
import jax.numpy as jnp
from jax import lax
from jax.experimental import pallas as pl
from jax.experimental.pallas import tpu as pltpu
from jax.experimental.pallas import tpu_sc as plsc

_N = 50000
_E = 1600000
_C = 32
_NPAD = 50176
_EPAD = 1638400
_NC = 2
_NS = 16
_NW = _NC * _NS
_EPT = _E // _NW
_CHUNK = 400
_NCHUNK = _EPT // _CHUNK
_SB = 100
_NB = _CHUNK // _SB
_ROWS_PT = _NPAD // _NS
_ZC = _ROWS_PT // 8

_BN = 512
_MROWS = 128


def _mlp_body(attr_ref, p_ref, a_ref, b_ref):
    def w(i):
        return p_ref[i : i + 1, 0:1]

    ak = [attr_ref[k] for k in range(4)]
    hs = []
    for j in range(10):
        s = ak[0] * w(j)
        for k in range(1, 4):
            s = s + ak[k] * w(k * 10 + j)
        hs.append(jnp.maximum(s + w(40 + j), 0.0))
    s0 = hs[0] * w(50)
    s1 = hs[0] * w(51)
    for j in range(1, 10):
        s0 = s0 + hs[j] * w(50 + 2 * j)
        s1 = s1 + hs[j] * w(51 + 2 * j)
    t0 = jnp.tanh(s0 + w(70))
    t1 = jnp.tanh(s1 + w(71))
    dd = w(72) * w(72)
    a_ref[...] = t0 + dd
    b_ref[...] = t1 - dd


@functools.cache
def _get_mlp_call():
  return pl.pallas_call(
    _mlp_body,
    grid=(_EPAD // (_MROWS * 1280),),
    in_specs=[
        pl.BlockSpec((4, _MROWS, 1280), lambda i: (0, i, 0)),
        pl.BlockSpec((73, 1), lambda i: (0, 0)),
    ],
    out_specs=[
        pl.BlockSpec((_MROWS, 1280), lambda i: (i, 0)),
        pl.BlockSpec((_MROWS, 1280), lambda i: (i, 0)),
    ],
    out_shape=[
        jax.ShapeDtypeStruct((_EPAD // 1280, 1280), jnp.float32),
        jax.ShapeDtypeStruct((_EPAD // 1280, 1280), jnp.float32),
    ],
  )


def _sc_body(xt_hbm, src_hbm, dst_hbm, a_hbm, b_hbm,
             acc_hbm, s_hbm,
             dstbuf, abuf, srcbuf, bbuf, rows, szbuf, acc_sp, s_sp,
             sem, gsem, ssem):
    cid = lax.axis_index("c")
    sid = lax.axis_index("s")
    wid = cid * _NS + sid

    zv = jnp.zeros((16,), jnp.float32)

    @plsc.parallel_loop(0, _CHUNK)
    def _(r):
        rows[r, 0:16] = zv
        rows[r, 16:32] = zv

    @plsc.parallel_loop(0, _ROWS_PT // 16)
    def _(i):
        szbuf[pl.ds(i * 16, 16)] = zv

    nbase = sid * _ROWS_PT
    for i in range(8):
        pltpu.sync_copy(rows.at[pl.ds(0, _ZC)],
                        acc_sp.at[pl.ds(nbase + i * _ZC, _ZC)])
    pltpu.sync_copy(szbuf, s_sp.at[pl.ds(nbase, _ROWS_PT)])
    plsc.subcore_barrier()

    def chunk_body(g, carry):
        ebase = pl.multiple_of(wid * _EPT + g * _CHUNK, _CHUNK)
        rbase = pl.multiple_of(wid * (_EPT // _SB) + g * _NB, _NB)
        cps = [
            pltpu.async_copy(dst_hbm.at[pl.ds(ebase, _CHUNK)], dstbuf, sem),
            pltpu.async_copy(a_hbm.at[pl.ds(ebase, _CHUNK)], abuf, sem),
            pltpu.async_copy(src_hbm.at[pl.ds(rbase, _NB)], srcbuf, sem),
            pltpu.async_copy(b_hbm.at[pl.ds(rbase, _NB)], bbuf, sem),
        ]
        for cp in cps:
            cp.wait()
        pltpu.async_copy(xt_hbm.at[dstbuf], rows, gsem).wait()

        @plsc.parallel_loop(0, _CHUNK // 16, unroll=2)
        def _(g16):
            e0 = g16 * 16
            av16 = abuf[pl.ds(e0, 16)]
            for j in range(16):
                e = e0 + j
                av = jnp.full((16,), av16[j], jnp.float32)
                rows[e, 0:16] = rows[e, 0:16] * av
                rows[e, 16:32] = rows[e, 16:32] * av

        scs = []
        for j in range(_NB):
            scs.append(pltpu.async_copy(
                rows.at[pl.ds(j * _SB, _SB)], acc_sp.at[srcbuf.at[j]],
                ssem, add=True))
            scs.append(pltpu.async_copy(
                bbuf.at[j], s_sp.at[srcbuf.at[j]], ssem, add=True))
        for cp in scs:
            cp.wait()
        return carry

    lax.fori_loop(0, _NCHUNK, chunk_body, 0)

    plsc.subcore_barrier()
    pltpu.sync_copy(acc_sp.at[pl.ds(nbase, _ROWS_PT)],
                    acc_hbm.at[cid, pl.ds(nbase, _ROWS_PT)])
    sobase = pl.multiple_of(cid * _NPAD + nbase, 8)
    pltpu.sync_copy(s_sp.at[pl.ds(nbase, _ROWS_PT)],
                    s_hbm.at[pl.ds(sobase, _ROWS_PT)])


@functools.cache
def _get_sc_call():
  return pl.kernel(
    _sc_body,
    out_type=(
        jax.ShapeDtypeStruct((_NC, _NPAD, _C), jnp.float32),
        jax.ShapeDtypeStruct((_NC * _NPAD,), jnp.float32),
    ),
    mesh=plsc.VectorSubcoreMesh(
        core_axis_name="c", subcore_axis_name="s",
        num_cores=_NC, num_subcores=_NS),
    compiler_params=pltpu.CompilerParams(use_tc_tiling_on_sc=False),
    scratch_types=[
        pltpu.VMEM((_CHUNK,), jnp.int32),
        pltpu.VMEM((_CHUNK,), jnp.float32),
        pltpu.VMEM((_NB, _SB), jnp.int32),
        pltpu.VMEM((_NB, _SB), jnp.float32),
        pltpu.VMEM((_CHUNK, _C), jnp.float32),
        pltpu.VMEM((_ROWS_PT,), jnp.float32),
        pltpu.VMEM_SHARED((_NPAD, _C), jnp.float32),
        pltpu.VMEM_SHARED((_NPAD,), jnp.float32),
        pltpu.SemaphoreType.DMA,
        pltpu.SemaphoreType.DMA,
        pltpu.SemaphoreType.DMA,
    ],
  )


def _comb_body(x_ref, acc_ref, s_ref, o_ref):
    asum = acc_ref[0] + acc_ref[1]
    att = asum.T
    scale = 1.0 + s_ref[0:1, :] + s_ref[1:2, :]
    o_ref[0] = x_ref[0] * scale + att


@functools.cache
def _get_comb_call():
  return pl.pallas_call(
    _comb_body,
    grid=(_NPAD // _BN,),
    in_specs=[
        pl.BlockSpec((1, _C, _BN), lambda i: (0, 0, i)),
        pl.BlockSpec((_NC, _BN, _C), lambda i: (0, i, 0)),
        pl.BlockSpec((_NC, _BN), lambda i: (0, i)),
    ],
    out_specs=pl.BlockSpec((1, _C, _BN), lambda i: (0, 0, i)),
    out_shape=jax.ShapeDtypeStruct((1, _C, _NPAD), jnp.float32),
  )


@jax.jit
def kernel(x, edge_index, edge_attr, W1, b1, W2, b2, diff_param):
    src = edge_index[0]
    dst = edge_index[1]
    p = jnp.concatenate(
        [W1.reshape(-1), b1, W2.reshape(-1), b2, diff_param]).reshape(73, 1)
    attr_t = jnp.pad(edge_attr, ((0, _EPAD - _E), (0, 0))).T.reshape(
        4, _EPAD // 1280, 1280)
    a2d, b2d_full = _get_mlp_call()(attr_t, p)
    a1 = a2d.reshape(-1)[:_E]
    b2d = b2d_full.reshape(-1)[:_E].reshape(_E // _SB, _SB)
    src2 = src.reshape(_E // _SB, _SB)
    xt = jnp.pad(x[0], ((0, 0), (0, _NPAD - _N))).T
    acc, s = _get_sc_call()(xt, src2, dst, a1, b2d)
    s = s.reshape(_NC, _NPAD)
    xp = jnp.pad(x, ((0, 0), (0, 0), (0, _NPAD - _N)))
    outp = _get_comb_call()(xp, acc, s)
    return outp[:, :, :_N]


# speedup vs baseline: 19.8424x; 1.2364x over previous
import functools

import jax
import jax.numpy as jnp
from jax import lax
from jax.experimental import pallas as pl
from jax.experimental.pallas import tpu as pltpu
from jax.experimental.pallas import tpu_sc as plsc

_N = 50000
_E = 1600000
_C = 32
_NPAD = 50176
_E2 = 1605632
_ER = _E2 // 128
_NC = 2
_NS = 16
_NW = _NC * _NS
_EPT = _E2 // _NW
_CHUNK = 256
_NCHUNK = _EPT // _CHUNK
_SB = 128
_NB = _CHUNK // _SB
_ROWS_PT = _NPAD // _NS
_ZC = 224
_SZ = 448

_BN = 512
_MR = _ER // 8
_EROWS = _E // 128


def _mlp_body(attr_ref, p_ref, a_ref, b_ref):
    i = pl.program_id(0)

    def w(k):
        return p_ref[k : k + 1, 0:1]

    ak = [attr_ref[k] for k in range(4)]
    hs = []
    for j in range(10):
        s = ak[0] * w(j)
        for k in range(1, 4):
            s = s + ak[k] * w(k * 10 + j)
        hs.append(jnp.maximum(s + w(40 + j), 0.0))
    s0 = hs[0] * w(50)
    s1 = hs[0] * w(51)
    for j in range(1, 10):
        s0 = s0 + hs[j] * w(50 + 2 * j)
        s1 = s1 + hs[j] * w(51 + 2 * j)
    t0 = jnp.tanh(s0 + w(70))
    t1 = jnp.tanh(s1 + w(71))
    dd = w(72) * w(72)
    rows = jax.lax.broadcasted_iota(jnp.int32, (_MR, 128), 0) + i * _MR
    live = rows < _EROWS
    a_ref[...] = jnp.where(live, t0 + dd, 0.0)
    b_ref[...] = jnp.where(live, t1 - dd, 0.0)


@functools.cache
def _get_mlp_call():
  return pl.pallas_call(
    _mlp_body,
    grid=(_ER // _MR,),
    in_specs=[
        pl.BlockSpec((4, _MR, 128), lambda i: (0, i, 0)),
        pl.BlockSpec((73, 1), lambda i: (0, 0)),
    ],
    out_specs=[
        pl.BlockSpec((_MR, 128), lambda i: (i, 0)),
        pl.BlockSpec((_MR, 128), lambda i: (i, 0)),
    ],
    out_shape=[
        jax.ShapeDtypeStruct((_ER, 128), jnp.float32),
        jax.ShapeDtypeStruct((_ER, 128), jnp.float32),
    ],
  )


def _sc_body(xt_hbm, src_hbm, dst_hbm, a_hbm, b_hbm,
             acc_hbm, s_hbm,
             dstbuf, abuf, srcbuf, bbuf, rows, szbuf, acc_sp, s_sp,
             seml, semg, sems):
    cid = lax.axis_index("c")
    sid = lax.axis_index("s")
    wid = cid * _NS + sid
    ept_rows = _EPT // _SB

    zv = jnp.zeros((16,), jnp.float32)

    @plsc.parallel_loop(0, _CHUNK)
    def _(r):
        rows[0][r, 0:16] = zv
        rows[0][r, 16:32] = zv

    @plsc.parallel_loop(0, _SZ // 16)
    def _(r):
        szbuf[pl.ds(r * 16, 16)] = zv

    nbase = sid * _ROWS_PT
    for k in range(_ROWS_PT // _ZC):
        pltpu.sync_copy(rows[0].at[pl.ds(0, _ZC)],
                        acc_sp.at[pl.ds(nbase + k * _ZC, _ZC)])
    for k in range(_ROWS_PT // _SZ):
        pltpu.sync_copy(szbuf, s_sp.at[pl.ds(nbase + k * _SZ, _SZ)])
    plsc.subcore_barrier()

    def issue_loads(g, t):
        ebase = pl.multiple_of(wid * _EPT + g * _CHUNK, _CHUNK)
        rbase = pl.multiple_of(wid * ept_rows + g * _NB, _NB)
        pltpu.async_copy(dst_hbm.at[pl.ds(ebase, _CHUNK)], dstbuf[t], seml[t])
        pltpu.async_copy(a_hbm.at[pl.ds(rbase, _NB)], abuf[t], seml[t])
        pltpu.async_copy(src_hbm.at[pl.ds(rbase, _NB)], srcbuf[t], seml[t])
        pltpu.async_copy(b_hbm.at[pl.ds(rbase, _NB)], bbuf[t], seml[t])

    def wait_loads(t):
        pltpu.make_async_copy(dst_hbm.at[pl.ds(0, _CHUNK)], dstbuf[t], seml[t]).wait()
        pltpu.make_async_copy(a_hbm.at[pl.ds(0, _NB)], abuf[t], seml[t]).wait()
        pltpu.make_async_copy(src_hbm.at[pl.ds(0, _NB)], srcbuf[t], seml[t]).wait()
        pltpu.make_async_copy(b_hbm.at[pl.ds(0, _NB)], bbuf[t], seml[t]).wait()

    def issue_gather(t):
        pltpu.async_copy(xt_hbm.at[dstbuf[t]], rows[t], semg[t])

    def wait_gather(t):
        pltpu.make_async_copy(xt_hbm.at[dstbuf[t]], rows[t], semg[t]).wait()

    def compute(t):
        @plsc.parallel_loop(0, _CHUNK // 16, unroll=2)
        def _(u):
            av16 = abuf[t][u // 8, pl.ds((u % 8) * 16, 16)]
            for j in range(16):
                e = u * 16 + j
                av = jnp.full((16,), av16[j], jnp.float32)
                rows[t][e, 0:16] = rows[t][e, 0:16] * av
                rows[t][e, 16:32] = rows[t][e, 16:32] * av

    def issue_scatter(t):
        for j in range(_NB):
            pltpu.async_copy(rows[t].at[pl.ds(j * _SB, _SB)],
                             acc_sp.at[srcbuf[t].at[j]], sems[t], add=True)
            pltpu.async_copy(bbuf[t].at[j], s_sp.at[srcbuf[t].at[j]],
                             sems[t], add=True)

    def wait_scatter(t):
        for j in range(_NB):
            pltpu.make_async_copy(rows[t].at[pl.ds(j * _SB, _SB)],
                                  acc_sp.at[srcbuf[t].at[j]], sems[t]).wait()
            pltpu.make_async_copy(bbuf[t].at[j], s_sp.at[srcbuf[t].at[j]],
                                  sems[t]).wait()

    issue_loads(0, 0)
    wait_loads(0)
    issue_gather(0)

    def stage(g, cur):
        oth = 1 - cur

        @pl.when(jnp.logical_and(g >= 2, g < _NCHUNK))
        def _():
            wait_scatter(cur)

        @pl.when(g < _NCHUNK)
        def _():
            issue_loads(g, cur)
        wait_gather(oth)
        compute(oth)
        issue_scatter(oth)

        @pl.when(g < _NCHUNK)
        def _():
            wait_loads(cur)
            issue_gather(cur)

    def pair(i, carry):
        stage(2 * i + 1, 1)
        stage(2 * i + 2, 0)
        return carry

    lax.fori_loop(0, _NCHUNK // 2, pair, 0)
    wait_scatter(0)
    wait_scatter(1)

    plsc.subcore_barrier()
    pltpu.sync_copy(acc_sp.at[pl.ds(nbase, _ROWS_PT)],
                    acc_hbm.at[cid, pl.ds(nbase, _ROWS_PT)])
    sobase = pl.multiple_of(cid * _NPAD + nbase, 8)
    pltpu.sync_copy(s_sp.at[pl.ds(nbase, _ROWS_PT)],
                    s_hbm.at[pl.ds(sobase, _ROWS_PT)])


@functools.cache
def _get_sc_call():
  return pl.kernel(
    _sc_body,
    out_type=(
        jax.ShapeDtypeStruct((_NC, _NPAD, _C), jnp.float32),
        jax.ShapeDtypeStruct((_NC * _NPAD,), jnp.float32),
    ),
    mesh=plsc.VectorSubcoreMesh(
        core_axis_name="c", subcore_axis_name="s",
        num_cores=_NC, num_subcores=_NS),
    compiler_params=pltpu.CompilerParams(use_tc_tiling_on_sc=False),
    scratch_types=[
        [pltpu.VMEM((_CHUNK,), jnp.int32)] * 2,
        [pltpu.VMEM((_NB, _SB), jnp.float32)] * 2,
        [pltpu.VMEM((_NB, _SB), jnp.int32)] * 2,
        [pltpu.VMEM((_NB, _SB), jnp.float32)] * 2,
        [pltpu.VMEM((_CHUNK, _C), jnp.float32)] * 2,
        pltpu.VMEM((_SZ,), jnp.float32),
        pltpu.VMEM_SHARED((_NPAD, _C), jnp.float32),
        pltpu.VMEM_SHARED((_NPAD,), jnp.float32),
        [pltpu.SemaphoreType.DMA] * 2,
        [pltpu.SemaphoreType.DMA] * 2,
        [pltpu.SemaphoreType.DMA] * 2,
    ],
  )


def _comb_body(x_ref, acc_ref, s0_ref, s1_ref, o_ref):
    asum = acc_ref[0] + acc_ref[1]
    eye = (jax.lax.broadcasted_iota(jnp.int32, (_C, _C), 0)
           == jax.lax.broadcasted_iota(jnp.int32, (_C, _C), 1)).astype(jnp.float32)
    att = jax.lax.dot_general(eye, asum, (((1,), (1,)), ((), ())),
                              preferred_element_type=jnp.float32)
    scale = 1.0 + (s0_ref[...] + s1_ref[...]).reshape(1, _BN)
    o_ref[0] = x_ref[0] * scale + att


@functools.cache
def _get_comb_call():
  return pl.pallas_call(
    _comb_body,
    grid=(_NPAD // _BN,),
    in_specs=[
        pl.BlockSpec((1, _C, _BN), lambda i: (0, 0, i)),
        pl.BlockSpec((_NC, _BN, _C), lambda i: (0, i, 0)),
        pl.BlockSpec((_BN,), lambda i: (i,)),
        pl.BlockSpec((_BN,), lambda i: (_NPAD // _BN + i,)),
    ],
    out_specs=pl.BlockSpec((1, _C, _BN), lambda i: (0, 0, i)),
    out_shape=jax.ShapeDtypeStruct((1, _C, _N), jnp.float32),
  )


@jax.jit
def kernel(x, edge_index, edge_attr, W1, b1, W2, b2, diff_param):
    src = edge_index[0]
    dst = edge_index[1]
    p = jnp.concatenate(
        [W1.reshape(-1), b1, W2.reshape(-1), b2, diff_param]).reshape(73, 1)
    attr_t = jnp.pad(edge_attr, ((0, _E2 - _E), (0, 0))).T.reshape(4, _ER, 128)
    a2d, b2d = _get_mlp_call()(attr_t, p)
    src2 = jnp.pad(src, (0, _E2 - _E)).reshape(_ER, 128)
    dst1 = jnp.pad(dst, (0, _E2 - _E))
    xt = x[0].T
    acc, s = _get_sc_call()(xt, src2, dst1, a2d, b2d)
    return _get_comb_call()(x, acc, s, s)


# speedup vs baseline: 20.4957x; 1.0329x over previous
import functools

import jax
import jax.numpy as jnp
from jax import lax
from jax.experimental import pallas as pl
from jax.experimental.pallas import tpu as pltpu
from jax.experimental.pallas import tpu_sc as plsc

_N = 50000
_E = 1600000
_C = 32
_NPAD = 50176
_E2 = 1605632
_ER = _E2 // 128
_NC = 2
_NS = 16
_NW = _NC * _NS
_EPT = _E2 // _NW
_CHUNK = 256
_NCHUNK = _EPT // _CHUNK
_SB = 128
_NB = _CHUNK // _SB
_ROWS_PT = _NPAD // _NS
_ZC = 224
_SZ = 448

_BN = 512
_MR = _ER // 8
_EROWS = _E // 128


def _mlp_body(attr_ref, p_ref, a_ref, b_ref):
    i = pl.program_id(0)

    def w(k):
        return p_ref[k : k + 1, 0:1]

    ak = [attr_ref[k] for k in range(4)]
    hs = []
    for j in range(10):
        s = ak[0] * w(j)
        for k in range(1, 4):
            s = s + ak[k] * w(k * 10 + j)
        hs.append(jnp.maximum(s + w(40 + j), 0.0))
    s0 = hs[0] * w(50)
    s1 = hs[0] * w(51)
    for j in range(1, 10):
        s0 = s0 + hs[j] * w(50 + 2 * j)
        s1 = s1 + hs[j] * w(51 + 2 * j)
    t0 = jnp.tanh(s0 + w(70))
    t1 = jnp.tanh(s1 + w(71))
    dd = w(72) * w(72)
    rows = jax.lax.broadcasted_iota(jnp.int32, (_MR, 128), 0) + i * _MR
    live = rows < _EROWS
    a_ref[...] = jnp.where(live, t0 + dd, 0.0)
    b_ref[...] = jnp.where(live, t1 - dd, 0.0)


@functools.cache
def _get_mlp_call():
  return pl.pallas_call(
    _mlp_body,
    grid=(_ER // _MR,),
    in_specs=[
        pl.BlockSpec((4, _MR, 128), lambda i: (0, i, 0)),
        pl.BlockSpec((73, 1), lambda i: (0, 0)),
    ],
    out_specs=[
        pl.BlockSpec((_MR, 128), lambda i: (i, 0)),
        pl.BlockSpec((_MR, 128), lambda i: (i, 0)),
    ],
    out_shape=[
        jax.ShapeDtypeStruct((_ER, 128), jnp.float32),
        jax.ShapeDtypeStruct((_ER, 128), jnp.float32),
    ],
  )


def _sc_body(xt_hbm, src_hbm, dst_hbm, a_hbm, b_hbm,
             acc_hbm, s_hbm,
             dstbuf, abuf, srcbuf, bbuf, rows, szbuf, acc_sp, s_sp,
             seml, semg, sems):
    cid = lax.axis_index("c")
    sid = lax.axis_index("s")
    wid = cid * _NS + sid
    ept_rows = _EPT // _SB

    zv = jnp.zeros((16,), jnp.float32)

    @plsc.parallel_loop(0, _CHUNK)
    def _(r):
        rows[0][r, 0:16] = zv
        rows[0][r, 16:32] = zv

    @plsc.parallel_loop(0, _SZ // 16)
    def _(r):
        szbuf[pl.ds(r * 16, 16)] = zv

    nbase = sid * _ROWS_PT
    for k in range(_ROWS_PT // _ZC):
        pltpu.sync_copy(rows[0].at[pl.ds(0, _ZC)],
                        acc_sp.at[pl.ds(nbase + k * _ZC, _ZC)])
    for k in range(_ROWS_PT // _SZ):
        pltpu.sync_copy(szbuf, s_sp.at[pl.ds(nbase + k * _SZ, _SZ)])
    plsc.subcore_barrier()

    def issue_loads(g, l):
        ebase = pl.multiple_of(wid * _EPT + g * _CHUNK, _CHUNK)
        rbase = pl.multiple_of(wid * ept_rows + g * _NB, _NB)
        pltpu.async_copy(dst_hbm.at[pl.ds(ebase, _CHUNK)], dstbuf[l], seml[l])
        pltpu.async_copy(a_hbm.at[pl.ds(rbase, _NB)], abuf[l], seml[l])
        pltpu.async_copy(src_hbm.at[pl.ds(rbase, _NB)], srcbuf[l], seml[l])
        pltpu.async_copy(b_hbm.at[pl.ds(rbase, _NB)], bbuf[l], seml[l])

    def wait_loads(l):
        pltpu.make_async_copy(dst_hbm.at[pl.ds(0, _CHUNK)], dstbuf[l], seml[l]).wait()
        pltpu.make_async_copy(a_hbm.at[pl.ds(0, _NB)], abuf[l], seml[l]).wait()
        pltpu.make_async_copy(src_hbm.at[pl.ds(0, _NB)], srcbuf[l], seml[l]).wait()
        pltpu.make_async_copy(b_hbm.at[pl.ds(0, _NB)], bbuf[l], seml[l]).wait()

    def issue_gather(t, l):
        pltpu.async_copy(xt_hbm.at[dstbuf[l]], rows[t], semg[t])

    def wait_gather(t, l):
        pltpu.make_async_copy(xt_hbm.at[dstbuf[l]], rows[t], semg[t]).wait()

    def compute(t, l):
        @plsc.parallel_loop(0, _CHUNK // 16, unroll=2)
        def _(u):
            av16 = abuf[l][u // 8, pl.ds((u % 8) * 16, 16)]
            for j in range(16):
                e = u * 16 + j
                av = jnp.full((16,), av16[j], jnp.float32)
                rows[t][e, 0:16] = rows[t][e, 0:16] * av
                rows[t][e, 16:32] = rows[t][e, 16:32] * av

    def issue_scatter(t, l):
        for j in range(_NB):
            pltpu.async_copy(rows[t].at[pl.ds(j * _SB, _SB)],
                             acc_sp.at[srcbuf[l].at[j]], sems[t], add=True)
            pltpu.async_copy(bbuf[l].at[j], s_sp.at[srcbuf[l].at[j]],
                             sems[t], add=True)

    def wait_scatter(t, l):
        for j in range(_NB):
            pltpu.make_async_copy(rows[t].at[pl.ds(j * _SB, _SB)],
                                  acc_sp.at[srcbuf[l].at[j]], sems[t]).wait()
            pltpu.make_async_copy(bbuf[l].at[j], s_sp.at[srcbuf[l].at[j]],
                                  sems[t]).wait()

    issue_loads(0, 0)
    wait_loads(0)
    issue_gather(0, 0)
    issue_loads(1, 1)

    def stage(g, cur, lg):
        oth = 1 - cur
        lprev = (lg - 1) % 4

        @pl.when(g >= 2)
        def _():
            wait_scatter(cur, (lg - 2) % 4)

        @pl.when(g < _NCHUNK)
        def _():
            wait_loads(lg)
            issue_gather(cur, lg)

        @pl.when(g + 1 < _NCHUNK)
        def _():
            issue_loads(g + 1, (lg + 1) % 4)
        wait_gather(oth, lprev)
        compute(oth, lprev)
        issue_scatter(oth, lprev)

    def quad(i, carry):
        g0 = 4 * i + 1
        stage(g0, 1, 1)
        stage(g0 + 1, 0, 2)
        stage(g0 + 2, 1, 3)
        stage(g0 + 3, 0, 0)
        return carry

    lax.fori_loop(0, _NCHUNK // 4, quad, 0)
    wait_scatter(1, (_NCHUNK - 1) % 4)

    plsc.subcore_barrier()
    pltpu.sync_copy(acc_sp.at[pl.ds(nbase, _ROWS_PT)],
                    acc_hbm.at[cid, pl.ds(nbase, _ROWS_PT)])
    sobase = pl.multiple_of(cid * _NPAD + nbase, 8)
    pltpu.sync_copy(s_sp.at[pl.ds(nbase, _ROWS_PT)],
                    s_hbm.at[pl.ds(sobase, _ROWS_PT)])


@functools.cache
def _get_sc_call():
  return pl.kernel(
    _sc_body,
    out_type=(
        jax.ShapeDtypeStruct((_NC, _NPAD, _C), jnp.float32),
        jax.ShapeDtypeStruct((_NC * _NPAD,), jnp.float32),
    ),
    mesh=plsc.VectorSubcoreMesh(
        core_axis_name="c", subcore_axis_name="s",
        num_cores=_NC, num_subcores=_NS),
    compiler_params=pltpu.CompilerParams(use_tc_tiling_on_sc=False),
    scratch_types=[
        [pltpu.VMEM((_CHUNK,), jnp.int32)] * 4,
        [pltpu.VMEM((_NB, _SB), jnp.float32)] * 4,
        [pltpu.VMEM((_NB, _SB), jnp.int32)] * 4,
        [pltpu.VMEM((_NB, _SB), jnp.float32)] * 4,
        [pltpu.VMEM((_CHUNK, _C), jnp.float32)] * 2,
        pltpu.VMEM((_SZ,), jnp.float32),
        pltpu.VMEM_SHARED((_NPAD, _C), jnp.float32),
        pltpu.VMEM_SHARED((_NPAD,), jnp.float32),
        [pltpu.SemaphoreType.DMA] * 4,
        [pltpu.SemaphoreType.DMA] * 2,
        [pltpu.SemaphoreType.DMA] * 2,
    ],
  )


def _xt_body(x_ref, o_ref):
    eye = (jax.lax.broadcasted_iota(jnp.int32, (_C, _C), 0)
           == jax.lax.broadcasted_iota(jnp.int32, (_C, _C), 1)).astype(jnp.float32)
    o_ref[...] = jax.lax.dot_general(x_ref[0], eye, (((0,), (0,)), ((), ())),
                                     preferred_element_type=jnp.float32)


@functools.cache
def _get_xt_call():
  return pl.pallas_call(
    _xt_body,
    grid=(_NPAD // _BN,),
    in_specs=[pl.BlockSpec((1, _C, _BN), lambda i: (0, 0, i))],
    out_specs=pl.BlockSpec((_BN, _C), lambda i: (i, 0)),
    out_shape=jax.ShapeDtypeStruct((_NPAD, _C), jnp.float32),
  )


def _comb_body(x_ref, acc_ref, s0_ref, s1_ref, o_ref):
    asum = acc_ref[0] + acc_ref[1]
    eye = (jax.lax.broadcasted_iota(jnp.int32, (_C, _C), 0)
           == jax.lax.broadcasted_iota(jnp.int32, (_C, _C), 1)).astype(jnp.float32)
    att = jax.lax.dot_general(eye, asum, (((1,), (1,)), ((), ())),
                              preferred_element_type=jnp.float32)
    scale = 1.0 + (s0_ref[...] + s1_ref[...]).reshape(1, _BN)
    o_ref[0] = x_ref[0] * scale + att


@functools.cache
def _get_comb_call():
  return pl.pallas_call(
    _comb_body,
    grid=(_NPAD // _BN,),
    in_specs=[
        pl.BlockSpec((1, _C, _BN), lambda i: (0, 0, i)),
        pl.BlockSpec((_NC, _BN, _C), lambda i: (0, i, 0)),
        pl.BlockSpec((_BN,), lambda i: (i,)),
        pl.BlockSpec((_BN,), lambda i: (_NPAD // _BN + i,)),
    ],
    out_specs=pl.BlockSpec((1, _C, _BN), lambda i: (0, 0, i)),
    out_shape=jax.ShapeDtypeStruct((1, _C, _N), jnp.float32),
  )


@jax.jit
def kernel(x, edge_index, edge_attr, W1, b1, W2, b2, diff_param):
    src = edge_index[0]
    dst = edge_index[1]
    p = jnp.concatenate(
        [W1.reshape(-1), b1, W2.reshape(-1), b2, diff_param]).reshape(73, 1)
    attr_t = jnp.pad(edge_attr, ((0, _E2 - _E), (0, 0))).T.reshape(4, _ER, 128)
    a2d, b2d = _get_mlp_call()(attr_t, p)
    src2 = jnp.pad(src, (0, _E2 - _E)).reshape(_ER, 128)
    dst1 = jnp.pad(dst, (0, _E2 - _E))
    xt = _get_xt_call()(x)
    acc, s = _get_sc_call()(xt, src2, dst1, a2d, b2d)
    return _get_comb_call()(x, acc, s, s)
